# baseline (speedup 1.0000x reference)
import dataclasses
import functools

import numpy as np
import jax
import jax.numpy as jnp
from jax import lax
from jax.experimental import pallas as pl
from jax.experimental.pallas import tpu as pltpu
from jax.experimental.pallas import tpu_sc as plsc

_NUM_LEVELS = 16
_EMBED_C = 2
_BASE_RES = 16
_DESIRED_RES = 2048
_SCALE = float(np.exp2(np.log2(_DESIRED_RES / _BASE_RES) / (_NUM_LEVELS - 1)))
_N = 262144


def _level_params():
    offs = 0
    out = []
    maxp = 2 ** 19
    for l in range(_NUM_LEVELS):
        res = int(np.ceil(_BASE_RES * _SCALE ** l))
        p = min(maxp, (res + 1) ** 3)
        p = int(np.ceil(p / 8) * 8)
        out.append((res, p, offs))
        offs += p
    return out, offs


_LEVELS, _TBL_ROWS = _level_params()
_PRIME_Y = np.int32(np.int64(2654435761) - (1 << 32))
_PRIME_Z = np.int32(805459861)

_NC, _NS = 2, 16
_NW = _NC * _NS
_PPW = _N // _NW
_P = 512
_G = 16


def _sc_encode_body(coords_hbm, table_hbm, feats_hbm,
                    coords_il, coords_v, frac_v, idx_v, vals_v, feats_v,
                    sem_in, sem_g, sem_out):
    cid = lax.axis_index("c")
    sid = lax.axis_index("s")
    wid = sid * _NC + cid
    base0 = wid * _PPW

    @pl.loop(0, _PPW, step=_P)
    def _chunk(ci):
        base = base0 + ci
        pltpu.async_copy(coords_hbm.at[pl.ds(base, _P), :],
                         coords_il, sem_in).wait()

        @pl.loop(0, _P, step=_G)
        def _deil(g):
            rowi = lax.iota(jnp.int32, _G) + g
            cold = jnp.zeros((_G,), jnp.int32)
            for d in range(3):
                coords_v[pl.ds(g + d * _P, _G)] = plsc.load_gather(
                    coords_il, [rowi, cold + np.int32(d)])

        for l, (res, m, off) in enumerate(_LEVELS):
            res_f = np.float32(res)
            po2 = (m & (m - 1)) == 0

            @pl.loop(0, _P, step=_G)
            def _grp(g):
                s = pl.ds(g, _G)
                pis = []
                for d in range(3):
                    v = coords_v[pl.ds(g + d * _P, _G)]
                    xn = (v + np.float32(1.0)) * np.float32(0.5)
                    pos = xn * res_f + np.float32(0.5)
                    pi = pos.astype(jnp.int32)
                    frac_v[d, s] = pos - pi.astype(jnp.float32)
                    pis.append(pi)
                xi, yi, zi = pis
                yp0 = yi * _PRIME_Y
                yp1 = yp0 + _PRIME_Y
                zp0 = zi * _PRIME_Z
                zp1 = zp0 + _PRIME_Z
                c = 0
                for xc in (xi, xi + np.int32(1)):
                    for yp in (yp0, yp1):
                        for zp in (zp0, zp1):
                            h = xc ^ yp ^ zp
                            if po2:
                                r = h & np.int32(m - 1)
                            else:
                                hs = lax.shift_right_logical(h, 8)
                                hf = hs.astype(jnp.float32) * np.float32(256.0)
                                q = (hf * np.float32(1.0 / m)).astype(jnp.int32)
                                r = h - q * np.int32(m)
                                r = jnp.where(r < 0, r + np.int32(m), r)
                                r = jnp.where(r >= np.int32(m), r - np.int32(m), r)
                            idx_v[pl.ds(g + c * _P, _G)] = r + np.int32(off)
                            c += 1

            gcps = [pltpu.async_copy(table_hbm.at[idx_v.at[pl.ds(c * _P, _P)]],
                                     vals_v.at[pl.ds(c * _P, _P)], sem_g)
                    for c in range(8)]
            for cp in gcps:
                cp.wait()

            @pl.loop(0, _P, step=_G)
            def _acc(g):
                s = pl.ds(g, _G)
                rowi = lax.iota(jnp.int32, _G) + g
                col0 = jnp.zeros((_G,), jnp.int32)
                col1 = col0 + np.int32(1)
                fx = frac_v[0, s]
                fy = frac_v[1, s]
                fz = frac_v[2, s]
                one = np.float32(1.0)
                wxs = (one - fx, fx)
                wys = (one - fy, fy)
                wzs = (one - fz, fz)
                acc0 = acc1 = None
                c = 0
                for icx in range(2):
                    for icy in range(2):
                        for icz in range(2):
                            rowc = rowi + np.int32(c * _P)
                            v0 = plsc.load_gather(vals_v, [rowc, col0])
                            v1 = plsc.load_gather(vals_v, [rowc, col1])
                            w = wxs[icx] * wys[icy] * wzs[icz]
                            t0 = w * v0
                            t1 = w * v1
                            acc0 = t0 if acc0 is None else acc0 + t0
                            acc1 = t1 if acc1 is None else acc1 + t1
                            c += 1
                feats_v[2 * l, s] = acc0
                feats_v[2 * l + 1, s] = acc1

        pltpu.async_copy(feats_v, feats_hbm.at[:, pl.ds(base, _P)],
                         sem_out).wait()


_SC_CP = pltpu.CompilerParams()
if "needs_layout_passes" in pltpu.CompilerParams.__dataclass_fields__:
    _SC_CP = dataclasses.replace(_SC_CP, needs_layout_passes=False)
if "use_tc_tiling_on_sc" in pltpu.CompilerParams.__dataclass_fields__:
    _SC_CP = dataclasses.replace(_SC_CP, use_tc_tiling_on_sc=False)

_sc_encode = functools.partial(
    pl.kernel,
    compiler_params=_SC_CP,
    out_type=jax.ShapeDtypeStruct((2 * _NUM_LEVELS, _N), jnp.float32),
    mesh=plsc.VectorSubcoreMesh(core_axis_name="c", subcore_axis_name="s",
                                num_cores=_NC, num_subcores=_NS),
    scratch_types=[
        pltpu.VMEM((_P, 3), jnp.float32),
        pltpu.VMEM((3 * _P,), jnp.float32),
        pltpu.VMEM((3, _P), jnp.float32),
        pltpu.VMEM((8 * _P,), jnp.int32),
        pltpu.VMEM((8 * _P, 2), jnp.float32),
        pltpu.VMEM((2 * _NUM_LEVELS, _P), jnp.float32),
        pltpu.SemaphoreType.DMA,
        pltpu.SemaphoreType.DMA,
        pltpu.SemaphoreType.DMA,
    ],
)(_sc_encode_body)


def _mlp_body(f_ref, w1_ref, b1_ref, w2_ref, b2_ref, w3_ref, b3_ref, o_ref):
    f = f_ref[...]
    dn = (((1,), (0,)), ((), ()))
    h = lax.dot_general(w1_ref[...], f, dn,
                        preferred_element_type=jnp.float32) + b1_ref[...]
    h = jnp.where(h >= 0, h, np.float32(0.01) * h)
    h = lax.dot_general(w2_ref[...], h, dn,
                        preferred_element_type=jnp.float32) + b2_ref[...]
    h = jnp.where(h >= 0, h, np.float32(0.01) * h)
    o_ref[...] = lax.dot_general(w3_ref[...], h, dn,
                                 preferred_element_type=jnp.float32) + b3_ref[...]


_MLP_B = 2048


def _mlp(feats, W1, b1, W2, b2, W3, b3):
    return pl.pallas_call(
        _mlp_body,
        grid=(_N // _MLP_B,),
        in_specs=[
            pl.BlockSpec((2 * _NUM_LEVELS, _MLP_B), lambda i: (0, i)),
            pl.BlockSpec((32, 32), lambda i: (0, 0)),
            pl.BlockSpec((32, 1), lambda i: (0, 0)),
            pl.BlockSpec((32, 32), lambda i: (0, 0)),
            pl.BlockSpec((32, 1), lambda i: (0, 0)),
            pl.BlockSpec((1, 32), lambda i: (0, 0)),
            pl.BlockSpec((1, 1), lambda i: (0, 0)),
        ],
        out_specs=pl.BlockSpec((1, _MLP_B), lambda i: (0, i)),
        out_shape=jax.ShapeDtypeStruct((1, _N), jnp.float32),
    )(feats, W1, b1.reshape(32, 1), W2, b2.reshape(32, 1),
      W3, b3.reshape(1, 1))


def kernel(inputcoords, embeddings, W1, b1, W2, b2, W3, b3):
    n = inputcoords.shape[0]
    assert n == _N and embeddings.shape[0] == _TBL_ROWS
    coords_t = inputcoords.reshape(n, 3)
    table = embeddings.reshape(_TBL_ROWS, _EMBED_C)
    feats = _sc_encode(coords_t, table)
    out = _mlp(feats, W1, b1, W2, b2, W3, b3)
    return out.reshape(n, 1, 1)


# speedup vs baseline: 4.1299x; 4.1299x over previous
import dataclasses
import functools

import numpy as np
import jax
import jax.numpy as jnp
from jax import lax
from jax.experimental import pallas as pl
from jax.experimental.pallas import tpu as pltpu
from jax.experimental.pallas import tpu_sc as plsc

_NUM_LEVELS = 16
_EMBED_C = 2
_BASE_RES = 16
_DESIRED_RES = 2048
_SCALE = float(np.exp2(np.log2(_DESIRED_RES / _BASE_RES) / (_NUM_LEVELS - 1)))
_N = 262144


def _level_params():
    offs = 0
    out = []
    maxp = 2 ** 19
    for l in range(_NUM_LEVELS):
        res = int(np.ceil(_BASE_RES * _SCALE ** l))
        p = min(maxp, (res + 1) ** 3)
        p = int(np.ceil(p / 8) * 8)
        out.append((res, p, offs))
        offs += p
    return out, offs


_LEVELS, _TBL_ROWS = _level_params()
_PRIME_Y = np.int32(np.int64(2654435761) - (1 << 32))
_PRIME_Z = np.int32(805459861)

_NC, _NS = 2, 16
_NW = _NC * _NS
_PPW = _N // _NW
_P = 512
_G = 16


_IC = 2048
_ITILE = int(np.ceil(_TBL_ROWS / (_NW * _IC))) * _IC
_TPAD = _ITILE * _NW


def _sc_interleave_body(c0_hbm, c1_hbm, t8_hbm, c0_v, c1_v, out_v,
                        sem_in, sem_out):
    cid = lax.axis_index("c")
    sid = lax.axis_index("s")
    wid = sid * _NC + cid
    base0 = wid * _ITILE

    @pl.loop(0, _ITILE, step=_IC)
    def _chunk(k):
        rows = base0 + k
        cpa = pltpu.async_copy(c0_hbm.at[pl.ds(rows, _IC)], c0_v, sem_in)
        cpb = pltpu.async_copy(c1_hbm.at[pl.ds(rows, _IC)], c1_v, sem_in)
        cpa.wait()
        cpb.wait()

        @pl.loop(0, _IC, step=_G)
        def _grp(g):
            rowi = lax.iota(jnp.int32, _G) + g
            col0 = jnp.zeros((_G,), jnp.int32)
            v0 = c0_v[pl.ds(g, _G)]
            v1 = c1_v[pl.ds(g, _G)]
            plsc.store_scatter(out_v, [rowi, col0], v0)
            plsc.store_scatter(out_v, [rowi, col0 + np.int32(1)], v1)

        pltpu.async_copy(out_v, t8_hbm.at[pl.ds(rows, _IC), :],
                         sem_out).wait()


def _sc_encode_body(coords_hbm, table_hbm, feats_hbm,
                    coords_il, coords_v, frac_v, idx_v, vals_v, feats_v,
                    sem_in, sem_g, sem_out):
    cid = lax.axis_index("c")
    sid = lax.axis_index("s")
    wid = sid * _NC + cid
    base0 = wid * _PPW

    @pl.loop(0, _PPW, step=_P)
    def _chunk(ci):
        base = base0 + ci
        pltpu.async_copy(coords_hbm.at[pl.ds(base, _P), :],
                         coords_il, sem_in).wait()

        @pl.loop(0, _P, step=_G)
        def _deil(g):
            rowi = lax.iota(jnp.int32, _G) + g
            cold = jnp.zeros((_G,), jnp.int32)
            for d in range(3):
                coords_v[pl.ds(g + d * _P, _G)] = plsc.load_gather(
                    coords_il, [rowi, cold + np.int32(d)])

        for l, (res, m, off) in enumerate(_LEVELS):
            res_f = np.float32(res)
            po2 = (m & (m - 1)) == 0

            @pl.loop(0, _P, step=_G)
            def _grp(g):
                s = pl.ds(g, _G)
                pis = []
                for d in range(3):
                    v = coords_v[pl.ds(g + d * _P, _G)]
                    xn = (v + np.float32(1.0)) * np.float32(0.5)
                    pos = xn * res_f + np.float32(0.5)
                    pi = pos.astype(jnp.int32)
                    frac_v[d, s] = pos - pi.astype(jnp.float32)
                    pis.append(pi)
                xi, yi, zi = pis
                yp0 = yi * _PRIME_Y
                yp1 = yp0 + _PRIME_Y
                zp0 = zi * _PRIME_Z
                zp1 = zp0 + _PRIME_Z
                c = 0
                for xc in (xi, xi + np.int32(1)):
                    for yp in (yp0, yp1):
                        for zp in (zp0, zp1):
                            h = xc ^ yp ^ zp
                            if po2:
                                r = h & np.int32(m - 1)
                            else:
                                hs = lax.shift_right_logical(h, 8)
                                hf = hs.astype(jnp.float32) * np.float32(256.0)
                                q = (hf * np.float32(1.0 / m)).astype(jnp.int32)
                                r = h - q * np.int32(m)
                                r = jnp.where(r < 0, r + np.int32(m), r)
                                r = jnp.where(r >= np.int32(m), r - np.int32(m), r)
                            idx_v[pl.ds(g + c * _P, _G)] = r + np.int32(off)
                            c += 1

            gcps = [pltpu.async_copy(table_hbm.at[idx_v.at[pl.ds(c * _P, _P)]],
                                     vals_v.at[pl.ds(c * _P, _P)], sem_g)
                    for c in range(8)]
            for cp in gcps:
                cp.wait()

            @pl.loop(0, _P, step=_G)
            def _acc(g):
                s = pl.ds(g, _G)
                rowi = lax.iota(jnp.int32, _G) + g
                col0 = jnp.zeros((_G,), jnp.int32)
                col1 = col0 + np.int32(1)
                fx = frac_v[0, s]
                fy = frac_v[1, s]
                fz = frac_v[2, s]
                one = np.float32(1.0)
                wxs = (one - fx, fx)
                wys = (one - fy, fy)
                wzs = (one - fz, fz)
                acc0 = acc1 = None
                c = 0
                for icx in range(2):
                    for icy in range(2):
                        for icz in range(2):
                            rowc = rowi + np.int32(c * _P)
                            v0 = plsc.load_gather(vals_v, [rowc, col0])
                            v1 = plsc.load_gather(vals_v, [rowc, col1])
                            w = wxs[icx] * wys[icy] * wzs[icz]
                            t0 = w * v0
                            t1 = w * v1
                            acc0 = t0 if acc0 is None else acc0 + t0
                            acc1 = t1 if acc1 is None else acc1 + t1
                            c += 1
                feats_v[2 * l, s] = acc0
                feats_v[2 * l + 1, s] = acc1

        pltpu.async_copy(feats_v, feats_hbm.at[:, pl.ds(base, _P)],
                         sem_out).wait()


_SC_CP = pltpu.CompilerParams()
if "needs_layout_passes" in pltpu.CompilerParams.__dataclass_fields__:
    _SC_CP = dataclasses.replace(_SC_CP, needs_layout_passes=False)
if "use_tc_tiling_on_sc" in pltpu.CompilerParams.__dataclass_fields__:
    _SC_CP = dataclasses.replace(_SC_CP, use_tc_tiling_on_sc=False)

_sc_interleave = functools.partial(
    pl.kernel,
    compiler_params=_SC_CP,
    out_type=jax.ShapeDtypeStruct((_TPAD, 8), jnp.float32),
    mesh=plsc.VectorSubcoreMesh(core_axis_name="c", subcore_axis_name="s",
                                num_cores=_NC, num_subcores=_NS),
    scratch_types=[
        pltpu.VMEM((_IC,), jnp.float32),
        pltpu.VMEM((_IC,), jnp.float32),
        pltpu.VMEM((_IC, 8), jnp.float32),
        pltpu.SemaphoreType.DMA,
        pltpu.SemaphoreType.DMA,
    ],
)(_sc_interleave_body)

_sc_encode = functools.partial(
    pl.kernel,
    compiler_params=_SC_CP,
    out_type=jax.ShapeDtypeStruct((2 * _NUM_LEVELS, _N), jnp.float32),
    mesh=plsc.VectorSubcoreMesh(core_axis_name="c", subcore_axis_name="s",
                                num_cores=_NC, num_subcores=_NS),
    scratch_types=[
        pltpu.VMEM((_P, 3), jnp.float32),
        pltpu.VMEM((3 * _P,), jnp.float32),
        pltpu.VMEM((3, _P), jnp.float32),
        pltpu.VMEM((8 * _P,), jnp.int32),
        pltpu.VMEM((8 * _P, 8), jnp.float32),
        pltpu.VMEM((2 * _NUM_LEVELS, _P), jnp.float32),
        pltpu.SemaphoreType.DMA,
        pltpu.SemaphoreType.DMA,
        pltpu.SemaphoreType.DMA,
    ],
)(_sc_encode_body)


def _mlp_body(f_ref, w1_ref, b1_ref, w2_ref, b2_ref, w3_ref, b3_ref, o_ref):
    f = f_ref[...]
    dn = (((1,), (0,)), ((), ()))
    h = lax.dot_general(w1_ref[...], f, dn,
                        preferred_element_type=jnp.float32) + b1_ref[...]
    h = jnp.where(h >= 0, h, np.float32(0.01) * h)
    h = lax.dot_general(w2_ref[...], h, dn,
                        preferred_element_type=jnp.float32) + b2_ref[...]
    h = jnp.where(h >= 0, h, np.float32(0.01) * h)
    o_ref[...] = lax.dot_general(w3_ref[...], h, dn,
                                 preferred_element_type=jnp.float32) + b3_ref[...]


_MLP_B = 2048


def _mlp(feats, W1, b1, W2, b2, W3, b3):
    return pl.pallas_call(
        _mlp_body,
        grid=(_N // _MLP_B,),
        in_specs=[
            pl.BlockSpec((2 * _NUM_LEVELS, _MLP_B), lambda i: (0, i)),
            pl.BlockSpec((32, 32), lambda i: (0, 0)),
            pl.BlockSpec((32, 1), lambda i: (0, 0)),
            pl.BlockSpec((32, 32), lambda i: (0, 0)),
            pl.BlockSpec((32, 1), lambda i: (0, 0)),
            pl.BlockSpec((1, 32), lambda i: (0, 0)),
            pl.BlockSpec((1, 1), lambda i: (0, 0)),
        ],
        out_specs=pl.BlockSpec((1, _MLP_B), lambda i: (0, i)),
        out_shape=jax.ShapeDtypeStruct((1, _N), jnp.float32),
    )(feats, W1, b1.reshape(32, 1), W2, b2.reshape(32, 1),
      W3, b3.reshape(1, 1))


def kernel(inputcoords, embeddings, W1, b1, W2, b2, W3, b3):
    n = inputcoords.shape[0]
    assert n == _N and embeddings.shape[0] == _TBL_ROWS
    coords_t = inputcoords.reshape(n, 3)
    ch0 = jnp.pad(embeddings[:, 0, 0], (0, _TPAD - _TBL_ROWS))
    ch1 = jnp.pad(embeddings[:, 0, 1], (0, _TPAD - _TBL_ROWS))
    table8 = _sc_interleave(ch0, ch1)
    feats = _sc_encode(coords_t, table8)
    out = _mlp(feats, W1, b1, W2, b2, W3, b3)
    return out.reshape(n, 1, 1)


# speedup vs baseline: 5.2536x; 1.2721x over previous
import dataclasses
import functools

import numpy as np
import jax
import jax.numpy as jnp
from jax import lax
from jax.experimental import pallas as pl
from jax.experimental.pallas import tpu as pltpu
from jax.experimental.pallas import tpu_sc as plsc

_NUM_LEVELS = 16
_EMBED_C = 2
_BASE_RES = 16
_DESIRED_RES = 2048
_SCALE = float(np.exp2(np.log2(_DESIRED_RES / _BASE_RES) / (_NUM_LEVELS - 1)))
_N = 262144


def _level_params():
    offs = 0
    out = []
    maxp = 2 ** 19
    for l in range(_NUM_LEVELS):
        res = int(np.ceil(_BASE_RES * _SCALE ** l))
        p = min(maxp, (res + 1) ** 3)
        p = int(np.ceil(p / 8) * 8)
        out.append((res, p, offs))
        offs += p
    return out, offs


_LEVELS, _TBL_ROWS = _level_params()
_PRIME_Y = np.int32(np.int64(2654435761) - (1 << 32))
_PRIME_Z = np.int32(805459861)

_NC, _NS = 2, 16
_NW = _NC * _NS
_PPW = _N // _NW
_P = 512
_G = 16


_IC = 2048
_ITILE = int(np.ceil(_TBL_ROWS / (_NW * _IC))) * _IC
_TPAD = _ITILE * _NW


def _sc_interleave_body(c0_hbm, c1_hbm, t8_hbm, c0_v, c1_v, out_v,
                        sem_in, sem_out):
    cid = lax.axis_index("c")
    sid = lax.axis_index("s")
    wid = sid * _NC + cid
    base0 = wid * _ITILE

    @pl.loop(0, _ITILE, step=_IC)
    def _chunk(k):
        rows = base0 + k
        cpa = pltpu.async_copy(c0_hbm.at[pl.ds(rows, _IC)], c0_v, sem_in)
        cpb = pltpu.async_copy(c1_hbm.at[pl.ds(rows, _IC)], c1_v, sem_in)
        cpa.wait()
        cpb.wait()

        @pl.loop(0, _IC, step=_G)
        def _grp(g):
            rowi = lax.iota(jnp.int32, _G) + g
            col0 = jnp.zeros((_G,), jnp.int32)
            v0 = c0_v[pl.ds(g, _G)]
            v1 = c1_v[pl.ds(g, _G)]
            plsc.store_scatter(out_v, [rowi, col0], v0)
            plsc.store_scatter(out_v, [rowi, col0 + np.int32(1)], v1)

        pltpu.async_copy(out_v, t8_hbm.at[pl.ds(rows, _IC), :],
                         sem_out).wait()


def _sc_encode_body(xs_hbm, ys_hbm, zs_hbm, table_hbm, feats_hbm,
                    coords_v, frac_v, idx_v, vals_v, feats_v,
                    sem_in, sem_g0, sem_g1, sem_out):
    cid = lax.axis_index("c")
    sid = lax.axis_index("s")
    wid = sid * _NC + cid
    base0 = wid * _PPW

    sem_gs = (sem_g0, sem_g1)

    @pl.loop(0, _PPW, step=_P)
    def _chunk(ci):
        base = base0 + ci
        cps = [pltpu.async_copy(ch.at[pl.ds(base, _P)],
                                coords_v.at[pl.ds(d * _P, _P)], sem_in)
               for d, ch in enumerate((xs_hbm, ys_hbm, zs_hbm))]
        for cp in cps:
            cp.wait()

        def stage_a(l, par):
            res, m, off = _LEVELS[l]
            res_f = np.float32(res)
            po2 = (m & (m - 1)) == 0
            fb = par * 3 * _P
            ib = par * 8 * _P

            @pl.loop(0, _P, step=_G)
            def _grp(g):
                pis = []
                for d in range(3):
                    v = coords_v[pl.ds(g + d * _P, _G)]
                    xn = (v + np.float32(1.0)) * np.float32(0.5)
                    pos = xn * res_f + np.float32(0.5)
                    pi = pos.astype(jnp.int32)
                    frac_v[pl.ds(g + fb + d * _P, _G)] = (
                        pos - pi.astype(jnp.float32))
                    pis.append(pi)
                xi, yi, zi = pis
                yp0 = yi * _PRIME_Y
                yp1 = yp0 + _PRIME_Y
                zp0 = zi * _PRIME_Z
                zp1 = zp0 + _PRIME_Z
                c = 0
                for xc in (xi, xi + np.int32(1)):
                    for yp in (yp0, yp1):
                        for zp in (zp0, zp1):
                            h = xc ^ yp ^ zp
                            if po2:
                                r = h & np.int32(m - 1)
                            else:
                                hs = lax.shift_right_logical(h, 8)
                                hf = hs.astype(jnp.float32) * np.float32(256.0)
                                q = (hf * np.float32(1.0 / m)).astype(jnp.int32)
                                r = h - q * np.int32(m)
                                r = jnp.where(r < 0, r + np.int32(m), r)
                                r = jnp.where(r >= np.int32(m), r - np.int32(m), r)
                            idx_v[pl.ds(g + ib + c * _P, _G)] = r + np.int32(off)
                            c += 1

        def fire(par):
            ib = par * 8 * _P
            return [pltpu.async_copy(
                table_hbm.at[idx_v.at[pl.ds(ib + c * _P, _P)]],
                vals_v.at[pl.ds(ib + c * _P, _P)], sem_gs[par])
                for c in range(8)]

        def stage_c(l, par):
            fb = par * 3 * _P
            ib = par * 8 * _P

            @pl.loop(0, _P, step=_G)
            def _acc(g):
                rowi = lax.iota(jnp.int32, _G) + (g + np.int32(ib))
                col0 = jnp.zeros((_G,), jnp.int32)
                col1 = col0 + np.int32(1)
                fx = frac_v[pl.ds(g + fb, _G)]
                fy = frac_v[pl.ds(g + fb + _P, _G)]
                fz = frac_v[pl.ds(g + fb + 2 * _P, _G)]
                one = np.float32(1.0)
                wxs = (one - fx, fx)
                wys = (one - fy, fy)
                wzs = (one - fz, fz)
                acc0 = acc1 = None
                c = 0
                for icx in range(2):
                    for icy in range(2):
                        for icz in range(2):
                            rowc = rowi + np.int32(c * _P)
                            v0 = plsc.load_gather(vals_v, [rowc, col0])
                            v1 = plsc.load_gather(vals_v, [rowc, col1])
                            w = wxs[icx] * wys[icy] * wzs[icz]
                            t0 = w * v0
                            t1 = w * v1
                            acc0 = t0 if acc0 is None else acc0 + t0
                            acc1 = t1 if acc1 is None else acc1 + t1
                            c += 1
                feats_v[2 * l, pl.ds(g, _G)] = acc0
                feats_v[2 * l + 1, pl.ds(g, _G)] = acc1

        stage_a(0, 0)
        gcps = fire(0)
        for l in range(_NUM_LEVELS):
            par = l % 2
            nxt = (l + 1) % 2
            if l + 1 < _NUM_LEVELS:
                stage_a(l + 1, nxt)
                ncps = fire(nxt)
            else:
                ncps = None
            for cp in gcps:
                cp.wait()
            stage_c(l, par)
            gcps = ncps

        pltpu.async_copy(feats_v, feats_hbm.at[:, pl.ds(base, _P)],
                         sem_out).wait()


_SC_CP = pltpu.CompilerParams()
if "needs_layout_passes" in pltpu.CompilerParams.__dataclass_fields__:
    _SC_CP = dataclasses.replace(_SC_CP, needs_layout_passes=False)
if "use_tc_tiling_on_sc" in pltpu.CompilerParams.__dataclass_fields__:
    _SC_CP = dataclasses.replace(_SC_CP, use_tc_tiling_on_sc=False)

_sc_interleave = functools.partial(
    pl.kernel,
    compiler_params=_SC_CP,
    out_type=jax.ShapeDtypeStruct((_TPAD, 8), jnp.float32),
    mesh=plsc.VectorSubcoreMesh(core_axis_name="c", subcore_axis_name="s",
                                num_cores=_NC, num_subcores=_NS),
    scratch_types=[
        pltpu.VMEM((_IC,), jnp.float32),
        pltpu.VMEM((_IC,), jnp.float32),
        pltpu.VMEM((_IC, 8), jnp.float32),
        pltpu.SemaphoreType.DMA,
        pltpu.SemaphoreType.DMA,
    ],
)(_sc_interleave_body)

_sc_encode = functools.partial(
    pl.kernel,
    compiler_params=_SC_CP,
    out_type=jax.ShapeDtypeStruct((2 * _NUM_LEVELS, _N), jnp.float32),
    mesh=plsc.VectorSubcoreMesh(core_axis_name="c", subcore_axis_name="s",
                                num_cores=_NC, num_subcores=_NS),
    scratch_types=[
        pltpu.VMEM((3 * _P,), jnp.float32),
        pltpu.VMEM((2 * 3 * _P,), jnp.float32),
        pltpu.VMEM((2 * 8 * _P,), jnp.int32),
        pltpu.VMEM((2 * 8 * _P, 8), jnp.float32),
        pltpu.VMEM((2 * _NUM_LEVELS, _P), jnp.float32),
        pltpu.SemaphoreType.DMA,
        pltpu.SemaphoreType.DMA,
        pltpu.SemaphoreType.DMA,
        pltpu.SemaphoreType.DMA,
    ],
)(_sc_encode_body)


def _mlp_body(f_ref, w1_ref, b1_ref, w2_ref, b2_ref, w3_ref, b3_ref, o_ref):
    f = f_ref[...]
    dn = (((1,), (0,)), ((), ()))
    h = lax.dot_general(w1_ref[...], f, dn,
                        preferred_element_type=jnp.float32) + b1_ref[...]
    h = jnp.where(h >= 0, h, np.float32(0.01) * h)
    h = lax.dot_general(w2_ref[...], h, dn,
                        preferred_element_type=jnp.float32) + b2_ref[...]
    h = jnp.where(h >= 0, h, np.float32(0.01) * h)
    o_ref[...] = lax.dot_general(w3_ref[...], h, dn,
                                 preferred_element_type=jnp.float32) + b3_ref[...]


_MLP_B = 2048


def _mlp(feats, W1, b1, W2, b2, W3, b3):
    return pl.pallas_call(
        _mlp_body,
        grid=(_N // _MLP_B,),
        in_specs=[
            pl.BlockSpec((2 * _NUM_LEVELS, _MLP_B), lambda i: (0, i)),
            pl.BlockSpec((32, 32), lambda i: (0, 0)),
            pl.BlockSpec((32, 1), lambda i: (0, 0)),
            pl.BlockSpec((32, 32), lambda i: (0, 0)),
            pl.BlockSpec((32, 1), lambda i: (0, 0)),
            pl.BlockSpec((1, 32), lambda i: (0, 0)),
            pl.BlockSpec((1, 1), lambda i: (0, 0)),
        ],
        out_specs=pl.BlockSpec((1, _MLP_B), lambda i: (0, i)),
        out_shape=jax.ShapeDtypeStruct((1, _N), jnp.float32),
    )(feats, W1, b1.reshape(32, 1), W2, b2.reshape(32, 1),
      W3, b3.reshape(1, 1))


def kernel(inputcoords, embeddings, W1, b1, W2, b2, W3, b3):
    n = inputcoords.shape[0]
    assert n == _N and embeddings.shape[0] == _TBL_ROWS
    xs = inputcoords[:, 0, 0]
    ys = inputcoords[:, 0, 1]
    zs = inputcoords[:, 0, 2]
    ch0 = jnp.pad(embeddings[:, 0, 0], (0, _TPAD - _TBL_ROWS))
    ch1 = jnp.pad(embeddings[:, 0, 1], (0, _TPAD - _TBL_ROWS))
    table8 = _sc_interleave(ch0, ch1)
    feats = _sc_encode(xs, ys, zs, table8)
    out = _mlp(feats, W1, b1, W2, b2, W3, b3)
    return out.reshape(n, 1, 1)
